# Initial kernel scaffold; baseline (speedup 1.0000x reference)
import jax
import jax.numpy as jnp
from jax.experimental import pallas as pl


def kernel(x, edge_index, edge_attr, W_mlpt, b_mlpt, gamma_mlpt, beta_mlpt, W_gate, b_gate, gamma_gate, beta_gate, gamma_node, beta_node):
    raise NotImplementedError("write your pallas kernel here")


# speedup vs baseline: 2.1361x; 2.1361x over previous
import functools

import jax
import jax.numpy as jnp
from jax import lax
from jax.experimental import pallas as pl
from jax.experimental.pallas import tpu as pltpu
from jax.experimental.pallas import tpu_sc as plsc

_NC = 2
_NS = 16
_NW = _NC * _NS
_LANES = 16
_EPS = 1e-5


def _proj_body(x_ref, ws_ref, wd_ref, p_ref, d_ref):
    xv = x_ref[...]
    p_ref[...] = jnp.dot(xv, ws_ref[...], preferred_element_type=jnp.float32)
    d_ref[...] = jnp.dot(xv, wd_ref[...], preferred_element_type=jnp.float32)


def _stats_body(ea_ref, g_ref, we_ref, sum_ref, sq_ref):
    y = g_ref[...] + jnp.dot(ea_ref[...], we_ref[...],
                             preferred_element_type=jnp.float32)
    s = jnp.sum(y, axis=0, keepdims=True)
    q = jnp.sum(y * y, axis=0, keepdims=True)

    @pl.when(pl.program_id(0) == 0)
    def _():
        sum_ref[...] = s
        sq_ref[...] = q

    @pl.when(pl.program_id(0) > 0)
    def _():
        sum_ref[...] += s
        sq_ref[...] += q


def _act_body(n_edges, d_feat, ea_ref, g_ref, we_ref, sum_ref, sq_ref,
              gam_ref, bet_ref, m_ref):
    inv_e = 1.0 / n_edges
    mean = sum_ref[...] * inv_e
    var = sq_ref[...] * inv_e - mean * mean
    inv = lax.rsqrt(var + _EPS)
    scale = gam_ref[...] * inv
    shift = bet_ref[...] - mean * scale
    y = g_ref[...] + jnp.dot(ea_ref[...], we_ref[...],
                             preferred_element_type=jnp.float32)
    z = y * scale + shift
    zm = z[:, :d_feat]
    zg = z[:, d_feat:]
    m_ref[...] = jax.nn.sigmoid(zm) * jax.nn.softplus(zg)


def _final_body(n_nodes, part_ref, x_ref, gn_ref, bn_ref, out_ref):
    agg = part_ref[0, :n_nodes, :] + part_ref[1, :n_nodes, :]
    mean = jnp.mean(agg, axis=0, keepdims=True)
    cent = agg - mean
    var = jnp.mean(cent * cent, axis=0, keepdims=True)
    z = cent * lax.rsqrt(var + _EPS) * gn_ref[...] + bn_ref[...]
    out_ref[...] = jax.nn.sigmoid(z + x_ref[...])


def kernel(x, edge_index, edge_attr, W_mlpt, b_mlpt, gamma_mlpt, beta_mlpt,
           W_gate, b_gate, gamma_gate, beta_gate, gamma_node, beta_node):
    del b_mlpt, b_gate
    n_nodes, d = x.shape
    n_edges = edge_attr.shape[0]
    d2 = 2 * d

    w_src = jnp.concatenate([W_mlpt[:d], W_gate[:d]], axis=1)
    w_dst = jnp.concatenate([W_mlpt[d:2 * d], W_gate[d:2 * d]], axis=1)
    w_edge = jnp.concatenate([W_mlpt[2 * d:], W_gate[2 * d:]], axis=1)
    gam = jnp.concatenate([gamma_mlpt, gamma_gate]).reshape(1, d2)
    bet = jnp.concatenate([beta_mlpt, beta_gate]).reshape(1, d2)
    src = edge_index[0]
    dst = edge_index[1]

    p_tab, d_tab = pl.pallas_call(
        _proj_body,
        out_shape=[jax.ShapeDtypeStruct((n_nodes, d2), jnp.float32),
                   jax.ShapeDtypeStruct((n_nodes, d2), jnp.float32)],
    )(x, w_src, w_dst)

    ew = n_edges // _NW
    blk = 80
    n_chunks = ew // blk
    mesh = plsc.VectorSubcoreMesh(core_axis_name="c", subcore_axis_name="s",
                                  num_cores=_NC, num_subcores=_NS)

    @functools.partial(
        pl.kernel,
        out_type=jax.ShapeDtypeStruct((n_edges, d2), jnp.float32),
        mesh=mesh,
        scratch_types=[
            pltpu.VMEM((blk,), jnp.int32),
            pltpu.VMEM((blk,), jnp.int32),
            pltpu.VMEM((blk, d2), jnp.float32),
            pltpu.VMEM((blk, d2), jnp.float32),
            pltpu.SemaphoreType.DMA,
            pltpu.SemaphoreType.DMA,
        ],
    )
    def _gather_add(p_hbm, dn_hbm, src_hbm, dst_hbm, g_hbm,
                    sidx, didx, prow, drow, sem_p, sem_d):
        wid = lax.axis_index("s") * _NC + lax.axis_index("c")
        base = wid * ew

        def chunk(i, carry):
            off = base + i * blk
            pltpu.sync_copy(src_hbm.at[pl.ds(off, blk)], sidx)
            pltpu.sync_copy(dst_hbm.at[pl.ds(off, blk)], didx)
            cp_p = pltpu.async_copy(p_hbm.at[sidx], prow, sem_p)
            cp_d = pltpu.async_copy(dn_hbm.at[didx], drow, sem_d)
            cp_p.wait()
            cp_d.wait()

            def row(r, c2):
                for j in range(d2 // _LANES):
                    sl = pl.ds(j * _LANES, _LANES)
                    drow[r, sl] = drow[r, sl] + prow[r, sl]
                return c2

            lax.fori_loop(0, blk, row, 0)
            pltpu.sync_copy(drow, g_hbm.at[pl.ds(off, blk)])
            return carry

        lax.fori_loop(0, n_chunks, chunk, 0)

    g = _gather_add(p_tab, d_tab, src, dst)

    be = 2000
    n_eblk = n_edges // be
    sums, sqs = pl.pallas_call(
        _stats_body,
        grid=(n_eblk,),
        in_specs=[
            pl.BlockSpec((be, d), lambda i: (i, 0)),
            pl.BlockSpec((be, d2), lambda i: (i, 0)),
            pl.BlockSpec((d, d2), lambda i: (0, 0)),
        ],
        out_specs=[pl.BlockSpec((1, d2), lambda i: (0, 0)),
                   pl.BlockSpec((1, d2), lambda i: (0, 0))],
        out_shape=[jax.ShapeDtypeStruct((1, d2), jnp.float32),
                   jax.ShapeDtypeStruct((1, d2), jnp.float32)],
    )(edge_attr, g, w_edge)

    m = pl.pallas_call(
        functools.partial(_act_body, float(n_edges), d),
        grid=(n_eblk,),
        in_specs=[
            pl.BlockSpec((be, d), lambda i: (i, 0)),
            pl.BlockSpec((be, d2), lambda i: (i, 0)),
            pl.BlockSpec((d, d2), lambda i: (0, 0)),
            pl.BlockSpec((1, d2), lambda i: (0, 0)),
            pl.BlockSpec((1, d2), lambda i: (0, 0)),
            pl.BlockSpec((1, d2), lambda i: (0, 0)),
            pl.BlockSpec((1, d2), lambda i: (0, 0)),
        ],
        out_specs=pl.BlockSpec((be, d), lambda i: (i, 0)),
        out_shape=jax.ShapeDtypeStruct((n_edges, d), jnp.float32),
    )(edge_attr, g, w_edge, sums, sqs, gam, bet)

    n_pad = ((n_nodes + 8 * _NS - 1) // (8 * _NS)) * (8 * _NS)
    rows_per_tile = n_pad // _NS

    @functools.partial(
        pl.kernel,
        out_type=jax.ShapeDtypeStruct((_NC, n_pad, d), jnp.float32),
        mesh=mesh,
        scratch_types=[
            pltpu.VMEM((blk,), jnp.int32),
            pltpu.VMEM((blk, d), jnp.float32),
            pltpu.VMEM((64, d), jnp.float32),
            pltpu.VMEM_SHARED((n_pad, d), jnp.float32),
            pltpu.SemaphoreType.DMA,
        ],
    )
    def _scatter_add(m_hbm, dst_hbm, out_hbm, didx, mrow, zbuf, agg_sh, sem):
        c = lax.axis_index("c")
        s = lax.axis_index("s")
        wid = s * _NC + c

        def zrow(r, carry):
            for j in range(d // _LANES):
                zbuf[r, pl.ds(j * _LANES, _LANES)] = jnp.zeros(
                    (_LANES,), jnp.float32)
            return carry

        lax.fori_loop(0, 64, zrow, 0)
        n_zc = (rows_per_tile + 63) // 64

        def zcopy(i, carry):
            off = jnp.minimum(i * 64, rows_per_tile - 64)
            pltpu.sync_copy(zbuf,
                            agg_sh.at[pl.ds(s * rows_per_tile + off, 64)])
            return carry

        lax.fori_loop(0, n_zc, zcopy, 0)
        plsc.subcore_barrier()

        base = wid * ew

        def chunk(i, carry):
            off = base + i * blk
            pltpu.sync_copy(dst_hbm.at[pl.ds(off, blk)], didx)
            pltpu.sync_copy(m_hbm.at[pl.ds(off, blk)], mrow)
            pltpu.sync_copy(mrow, agg_sh.at[didx], add=True)
            return carry

        lax.fori_loop(0, n_chunks, chunk, 0)
        plsc.subcore_barrier()
        pltpu.sync_copy(
            agg_sh.at[pl.ds(s * rows_per_tile, rows_per_tile)],
            out_hbm.at[c, pl.ds(s * rows_per_tile, rows_per_tile)])

    partials = _scatter_add(m, dst)

    node_out = pl.pallas_call(
        functools.partial(_final_body, n_nodes),
        out_shape=jax.ShapeDtypeStruct((n_nodes, d), jnp.float32),
    )(partials, x, gamma_node.reshape(1, d), beta_node.reshape(1, d))

    return (node_out, edge_attr)
